# Initial kernel scaffold; baseline (speedup 1.0000x reference)
import functools

import jax
import jax.numpy as jnp
from jax import lax
from jax.experimental import pallas as pl
from jax.experimental.pallas import tpu as pltpu
from jax.experimental.pallas import tpu_sc as plsc

N, E, DIN, DOUT = 10000, 160000, 256, 256
HALF = DOUT // 2
NC, NS, L = 2, 16, 16
PER_TEC = E // NS
CH = 80
NCH = PER_TEC // CH
RPT = N // NS
ZR = 125


_BN = 2000


def _matmul_body(x_ref, w_ref, b_ref, h_ref):
    h_ref[...] = (
        jnp.dot(x_ref[...], w_ref[...], preferred_element_type=jnp.float32)
        + b_ref[...]
    )


def _project(x, W, b2):
    return pl.pallas_call(
        _matmul_body,
        grid=(NC, N // _BN),
        in_specs=[
            pl.BlockSpec((_BN, DIN), lambda h, i: (i, 0)),
            pl.BlockSpec((DIN, HALF), lambda h, i: (0, h)),
            pl.BlockSpec((1, HALF), lambda h, i: (0, h)),
        ],
        out_specs=pl.BlockSpec((_BN, HALF), lambda h, i: (h * (N // _BN) + i, 0)),
        out_shape=jax.ShapeDtypeStruct((NC * N, HALF), jnp.float32),
    )(x, W, b2)


_sc_mesh = plsc.VectorSubcoreMesh(core_axis_name="c", subcore_axis_name="s")


@functools.partial(
    pl.kernel,
    mesh=_sc_mesh,
    out_type=jax.ShapeDtypeStruct((N, DOUT), jnp.float32),
    scratch_types=[
        pltpu.VMEM((PER_TEC,), jnp.int32),
        pltpu.VMEM((PER_TEC,), jnp.int32),
        pltpu.VMEM((PER_TEC,), jnp.float32),
        pltpu.VMEM((CH,), jnp.int32),
        pltpu.VMEM((CH, HALF), jnp.float32),
        pltpu.VMEM((ZR, HALF), jnp.float32),
        pltpu.VMEM_SHARED((N, HALF), jnp.float32),
        pltpu.SemaphoreType.DMA,
    ],
)
def _sc_aggregate(h2, col, row, ew, out, cbig, rbig, wbig, rbuf, gbuf, zbuf,
                  acc, sem):
    c = lax.axis_index("c")
    s = lax.axis_index("s")
    zeros = jnp.zeros((L,), jnp.float32)

    @pl.loop(0, ZR)
    def _zero(r):
        for j in range(HALF // L):
            zbuf[r, pl.ds(j * L, L)] = zeros

    for k in range(RPT // ZR):
        pltpu.sync_copy(zbuf, acc.at[pl.ds(s * RPT + k * ZR, ZR)])

    base = s * PER_TEC
    pltpu.sync_copy(col.at[pl.ds(base, PER_TEC)], cbig)
    pltpu.sync_copy(row.at[pl.ds(base, PER_TEC)], rbig)
    pltpu.sync_copy(ew.at[pl.ds(base, PER_TEC)], wbig)

    off = c * N

    @pl.loop(0, PER_TEC // L)
    def _bias(k):
        o = pl.multiple_of(k * L, L)
        cbig[pl.ds(o, L)] = cbig[pl.ds(o, L)] + off

    plsc.subcore_barrier()

    @pl.loop(0, NCH)
    def _chunk(i):
        e0 = pl.multiple_of(i * CH, L)
        pltpu.async_copy(h2.at[cbig.at[pl.ds(e0, CH)]], gbuf, sem).wait()

        @pl.loop(0, CH)
        def _scale(e):
            wv = plsc.load_gather(wbig, [jnp.full((L,), e0 + e, jnp.int32)])
            for j in range(HALF // L):
                gbuf[e, pl.ds(j * L, L)] = gbuf[e, pl.ds(j * L, L)] * wv

        for k in range(CH // L):
            o = pl.multiple_of(k * L, L)
            rbuf[pl.ds(o, L)] = rbig[pl.ds(e0 + o, L)]

        pltpu.sync_copy(gbuf, acc.at[rbuf], add=True)

    plsc.subcore_barrier()

    for k in range(RPT // ZR):
        r0 = s * RPT + k * ZR
        pltpu.sync_copy(
            acc.at[pl.ds(r0, ZR)],
            out.at[pl.ds(r0, ZR), pl.ds(c * HALF, HALF)],
        )


def kernel(x, edge_index, edge_weight, W, b):
    h2 = _project(x, W, b.reshape(1, DOUT))
    return _sc_aggregate(h2, edge_index[1], edge_index[0], edge_weight)


# speedup vs baseline: 4.3816x; 4.3816x over previous
import functools

import jax
import jax.numpy as jnp
from jax import lax
from jax.experimental import pallas as pl
from jax.experimental.pallas import tpu as pltpu
from jax.experimental.pallas import tpu_sc as plsc

N, E, DIN, DOUT = 10000, 160000, 256, 256
HALF = DOUT // 2
NC, NS, L = 2, 16, 16
PER_TEC = E // NS
CH = 80
STG = 2000
NSTG = PER_TEC // STG
NCH = STG // CH
RPT = 624
RPT_LAST = N - 15 * RPT


_BN = 2000


def _matmul_body(x_ref, w_ref, b_ref, h_ref):
    h_ref[...] = (
        jnp.dot(x_ref[...], w_ref[...], preferred_element_type=jnp.float32)
        + b_ref[...]
    )


def _project(x, W, b2):
    return pl.pallas_call(
        _matmul_body,
        grid=(NC, N // _BN),
        in_specs=[
            pl.BlockSpec((_BN, DIN), lambda h, i: (i, 0)),
            pl.BlockSpec((DIN, HALF), lambda h, i: (0, h)),
            pl.BlockSpec((1, HALF), lambda h, i: (0, h)),
        ],
        out_specs=pl.BlockSpec((_BN, HALF), lambda h, i: (h * (N // _BN) + i, 0)),
        out_shape=jax.ShapeDtypeStruct((NC * N, HALF), jnp.float32),
    )(x, W, b2)


_sc_mesh = plsc.VectorSubcoreMesh(core_axis_name="c", subcore_axis_name="s")


@functools.partial(
    pl.kernel,
    mesh=_sc_mesh,
    out_type=jax.ShapeDtypeStruct((N, DOUT), jnp.float32),
    scratch_types=[
        pltpu.VMEM((STG,), jnp.int32),
        pltpu.VMEM((STG,), jnp.int32),
        pltpu.VMEM((STG,), jnp.float32),
        pltpu.VMEM((CH,), jnp.int32),
        pltpu.VMEM((CH, HALF), jnp.float32),
        pltpu.VMEM_SHARED((N, HALF), jnp.float32),
        pltpu.SemaphoreType.DMA,
    ],
)
def _sc_aggregate(h2, col, row, ew, out, cbig, rbig, wbig, rbuf, gbuf,
                  acc, sem):
    c = lax.axis_index("c")
    s = lax.axis_index("s")
    zeros = jnp.zeros((L,), jnp.float32)

    @pl.loop(0, CH)
    def _zero(r):
        for j in range(HALF // L):
            gbuf[r, pl.ds(j * L, L)] = zeros

    r0 = pl.multiple_of(s * RPT, 8)
    for k in range(RPT // CH):
        pltpu.sync_copy(gbuf, acc.at[pl.ds(r0 + k * CH, CH)])
    rem = RPT - (RPT // CH) * CH
    pltpu.sync_copy(gbuf.at[pl.ds(0, rem)],
                    acc.at[pl.ds(r0 + (RPT // CH) * CH, rem)])

    @pl.when(s == NS - 1)
    def _zero_tail():
        pltpu.sync_copy(
            gbuf.at[pl.ds(0, RPT_LAST - RPT)],
            acc.at[pl.ds(r0 + RPT, RPT_LAST - RPT)],
        )

    plsc.subcore_barrier()

    off = c * N

    @pl.loop(0, NSTG)
    def _stage(g):
        base = pl.multiple_of(s * PER_TEC + g * STG, 8)
        pltpu.sync_copy(col.at[pl.ds(base, STG)], cbig)
        pltpu.sync_copy(row.at[pl.ds(base, STG)], rbig)
        pltpu.sync_copy(ew.at[pl.ds(base, STG)], wbig)

        @pl.loop(0, STG // L)
        def _bias(k):
            o = pl.multiple_of(k * L, L)
            cbig[pl.ds(o, L)] = cbig[pl.ds(o, L)] + off

        @pl.loop(0, NCH)
        def _chunk(i):
            e0 = pl.multiple_of(i * CH, L)
            pltpu.async_copy(h2.at[cbig.at[pl.ds(e0, CH)]], gbuf, sem).wait()

            @pl.loop(0, CH // L)
            def _scale(m):
                wv16 = wbig[pl.ds(e0 + pl.multiple_of(m * L, L), L)]
                for t in range(L):
                    wv = jnp.full((L,), wv16[t])
                    e = m * L + t
                    for j in range(HALF // L):
                        gbuf[e, pl.ds(j * L, L)] = (
                            gbuf[e, pl.ds(j * L, L)] * wv
                        )

            for k in range(CH // L):
                o = pl.multiple_of(k * L, L)
                rbuf[pl.ds(o, L)] = rbig[pl.ds(e0 + o, L)]

            pltpu.sync_copy(gbuf, acc.at[rbuf], add=True)

    plsc.subcore_barrier()

    c0 = pl.multiple_of(c * HALF, HALF)

    @pl.when(s < NS - 1)
    def _write_body():
        pltpu.sync_copy(
            acc.at[pl.ds(r0, RPT)],
            out.at[pl.ds(r0, RPT), pl.ds(c0, HALF)],
        )

    @pl.when(s == NS - 1)
    def _write_tail():
        pltpu.sync_copy(
            acc.at[pl.ds(r0, RPT_LAST)],
            out.at[pl.ds(r0, RPT_LAST), pl.ds(c0, HALF)],
        )


def kernel(x, edge_index, edge_weight, W, b):
    h2 = _project(x, W, b.reshape(1, DOUT))
    return _sc_aggregate(h2, edge_index[1], edge_index[0], edge_weight)


# speedup vs baseline: 8.3880x; 1.9144x over previous
import functools

import jax
import jax.numpy as jnp
from jax import lax
from jax.experimental import pallas as pl
from jax.experimental.pallas import tpu as pltpu
from jax.experimental.pallas import tpu_sc as plsc

N, E, DIN, DOUT = 10000, 160000, 256, 256
HALF = DOUT // 2
NC, NS, L = 2, 16, 16
PER_TEC = E // NS
CH = 80
STG = 2000
NSTG = PER_TEC // STG
NCH = STG // CH
RPT = 624
RPT_LAST = N - 15 * RPT


_BN = 2000


def _matmul_body(x_ref, w_ref, b_ref, h_ref):
    h_ref[...] = (
        jnp.dot(x_ref[...], w_ref[...], preferred_element_type=jnp.float32)
        + b_ref[...]
    )


def _project(x, W, b2):
    return pl.pallas_call(
        _matmul_body,
        grid=(NC, N // _BN),
        in_specs=[
            pl.BlockSpec((_BN, DIN), lambda h, i: (i, 0)),
            pl.BlockSpec((DIN, HALF), lambda h, i: (0, h)),
            pl.BlockSpec((1, HALF), lambda h, i: (0, h)),
        ],
        out_specs=pl.BlockSpec((_BN, HALF), lambda h, i: (h * (N // _BN) + i, 0)),
        out_shape=jax.ShapeDtypeStruct((NC * N, HALF), jnp.float32),
    )(x, W, b2)


_sc_mesh = plsc.VectorSubcoreMesh(core_axis_name="c", subcore_axis_name="s")


NB = 4
NTOT = PER_TEC // CH
NQ = (NTOT - 1) // NB


@functools.partial(
    pl.kernel,
    mesh=_sc_mesh,
    out_type=jax.ShapeDtypeStruct((N, DOUT), jnp.float32),
    scratch_types=(
        [pltpu.VMEM((CH,), jnp.int32) for _ in range(NB)]
        + [pltpu.VMEM((CH,), jnp.int32) for _ in range(NB)]
        + [pltpu.VMEM((CH,), jnp.float32) for _ in range(NB)]
        + [pltpu.VMEM((CH, HALF), jnp.float32) for _ in range(NB)]
        + [pltpu.VMEM_SHARED((N, HALF), jnp.float32)]
        + [pltpu.SemaphoreType.DMA for _ in range(3 * NB)]
    ),
)
def _sc_aggregate(h2, col, row, ew, out, *sc):
    cb, rb, wb, gb = sc[0:NB], sc[NB:2 * NB], sc[2 * NB:3 * NB], sc[3 * NB:4 * NB]
    acc = sc[4 * NB]
    gsem = sc[4 * NB + 1:4 * NB + 1 + NB]
    isem = sc[4 * NB + 1 + NB:4 * NB + 1 + 2 * NB]
    ssem = sc[4 * NB + 1 + 2 * NB:4 * NB + 1 + 3 * NB]

    c = lax.axis_index("c")
    s = lax.axis_index("s")
    zeros = jnp.zeros((L,), jnp.float32)
    off = c * N

    def idx_issue(i, b):
        base = pl.multiple_of(s * PER_TEC + i * CH, 8)
        pltpu.async_copy(col.at[pl.ds(base, CH)], cb[b], isem[b])
        pltpu.async_copy(row.at[pl.ds(base, CH)], rb[b], isem[b])
        pltpu.async_copy(ew.at[pl.ds(base, CH)], wb[b], isem[b])

    def idx_wait_bias(b):
        pltpu.make_async_copy(col.at[pl.ds(0, CH)], cb[b], isem[b]).wait()
        pltpu.make_async_copy(row.at[pl.ds(0, CH)], rb[b], isem[b]).wait()
        pltpu.make_async_copy(ew.at[pl.ds(0, CH)], wb[b], isem[b]).wait()
        for k in range(CH // L):
            o = pl.multiple_of(k * L, L)
            cb[b][pl.ds(o, L)] = cb[b][pl.ds(o, L)] + off

    def gather_issue(b):
        pltpu.async_copy(h2.at[cb[b]], gb[b], gsem[b])

    def gather_wait(b):
        pltpu.make_async_copy(h2.at[cb[b]], gb[b], gsem[b]).wait()

    def scatter_issue(b):
        pltpu.async_copy(gb[b], acc.at[rb[b]], ssem[b], add=True)

    def scatter_wait(b):
        pltpu.make_async_copy(gb[b], acc.at[rb[b]], ssem[b]).wait()

    def scale(b):
        gbuf, wbuf = gb[b], wb[b]

        @pl.loop(0, CH // L)
        def _scale(m):
            wv16 = wbuf[pl.ds(pl.multiple_of(m * L, L), L)]
            for t in range(L):
                wv = jnp.full((L,), wv16[t])
                e = m * L + t
                for j in range(HALF // L):
                    gbuf[e, pl.ds(j * L, L)] = gbuf[e, pl.ds(j * L, L)] * wv

    @pl.loop(0, CH)
    def _zero(r):
        for j in range(HALF // L):
            gb[0][r, pl.ds(j * L, L)] = zeros

    r0 = pl.multiple_of(s * RPT, 8)
    for k in range(RPT // CH):
        pltpu.sync_copy(gb[0], acc.at[pl.ds(r0 + k * CH, CH)])
    rem = RPT - (RPT // CH) * CH
    pltpu.sync_copy(gb[0].at[pl.ds(0, rem)],
                    acc.at[pl.ds(r0 + (RPT // CH) * CH, rem)])

    @pl.when(s == NS - 1)
    def _zero_tail():
        pltpu.sync_copy(
            gb[0].at[pl.ds(0, RPT_LAST - RPT)],
            acc.at[pl.ds(r0 + RPT, RPT_LAST - RPT)],
        )

    plsc.subcore_barrier()

    for i in range(NB - 1):
        idx_issue(i, i)
    for i in range(NB - 2):
        idx_wait_bias(i)
        gather_issue(i)

    @pl.loop(0, NQ)
    def _quad(q):
        for r in range(NB):
            i = q * NB + r
            b, b3, b2 = r, (r + 3) % NB, (r + 2) % NB
            gather_wait(b)
            scale(b)
            scatter_issue(b)

            if r == 0:
                @pl.when(q > 0)
                def _w():
                    scatter_wait(b3)
            else:
                scatter_wait(b3)

            @pl.when(i + NB - 1 <= NTOT - 1)
            def _i3():
                idx_issue(i + NB - 1, b3)

            @pl.when(i + NB - 2 <= NTOT - 1)
            def _i2():
                idx_wait_bias(b2)
                gather_issue(b2)

    lb = (NTOT - 1) % NB
    scatter_wait((lb + 3) % NB)
    gather_wait(lb)
    scale(lb)
    scatter_issue(lb)
    scatter_wait(lb)

    plsc.subcore_barrier()

    c0 = pl.multiple_of(c * HALF, HALF)

    @pl.when(s < NS - 1)
    def _write_body():
        pltpu.sync_copy(
            acc.at[pl.ds(r0, RPT)],
            out.at[pl.ds(r0, RPT), pl.ds(c0, HALF)],
        )

    @pl.when(s == NS - 1)
    def _write_tail():
        pltpu.sync_copy(
            acc.at[pl.ds(r0, RPT_LAST)],
            out.at[pl.ds(r0, RPT_LAST), pl.ds(c0, HALF)],
        )


def kernel(x, edge_index, edge_weight, W, b):
    h2 = _project(x, W, b.reshape(1, DOUT))
    return _sc_aggregate(h2, edge_index[1], edge_index[0], edge_weight)
